# baseline (speedup 1.0000x reference)
import jax
import jax.numpy as jnp
from jax import lax
from jax.experimental import pallas as pl
from jax.experimental.pallas import tpu as pltpu

N_B = 128
N_P = 55
PAD = 64
H = 64
L = 4
PAIRS = N_B // 2


def _silu_h(u):
    t = lax.tanh(u)
    return u * t + u


def _egnn_step(t_ref, x_ref,
               embw_ref, embb_ref,
               w0h_ref, w0c_ref, wra_ref, b0_ref,
               w1_ref, b1_ref,
               wc0_ref, bc0_ref, wc1_ref,
               wn0h_ref, wn0g_ref, bn0_ref, wn1_ref, bn1_ref,
               s12_ref, pij_ref, jm_ref,
               out_ref):
    f32 = jnp.float32
    x0 = x_ref[0]
    ta = t_ref[0, 0, 0]
    tb = t_ref[0, 0, 1]

    lane = lax.broadcasted_iota(jnp.int32, (1, 2 * H), 1)
    t_row = jnp.where(lane < H, ta, tb).astype(f32)
    h_row = t_row * embw_ref[...] + embb_ref[...]
    h = jnp.broadcast_to(h_row, (PAD, 2 * H))

    nmask = (lax.broadcasted_iota(jnp.int32, (PAD, 1), 0) < N_P).astype(f32)

    S12 = s12_ref[...]
    PIJ = pij_ref[...]
    JM = jm_ref[...]

    x = x0
    for l in range(L):
        diff = (x[:, None, :] - x[None, :, :]).reshape(PAD * PAD, 12)
        dsq = diff * diff
        radial = jnp.dot(dsq, S12, preferred_element_type=f32)
        inv = lax.rsqrt(radial + 1e-8) * JM

        A = jnp.dot(h, w0h_ref[l], preferred_element_type=f32) + b0_ref[l]
        B = jnp.dot(h, w0c_ref[l], preferred_element_type=f32)
        AB = jnp.concatenate([A, B], axis=0)
        pre0 = (jnp.dot(PIJ, AB, preferred_element_type=f32)
                + jnp.dot(dsq, wra_ref[l], preferred_element_type=f32))
        e0 = _silu_h(pre0)
        ef = _silu_h(jnp.dot(e0, w1_ref[l], preferred_element_type=f32)
                     + b1_ref[l])

        c0 = _silu_h(jnp.dot(ef, wc0_ref[l], preferred_element_type=f32)
                     + bc0_ref[l])
        s12 = jnp.dot(c0, wc1_ref[l], preferred_element_type=f32)
        trans = diff * (s12 * inv)
        upd = jnp.sum(trans.reshape(PAD, PAD, 12), axis=1)
        x = x + upd

        if l < L - 1:
            hagg = jnp.sum(ef.reshape(PAD, PAD, 2 * H)[:, :N_P, :], axis=1)
            e0d = _silu_h(A + B)
            efd = _silu_h(jnp.dot(e0d, w1_ref[l], preferred_element_type=f32)
                          + b1_ref[l])
            hagg = hagg - efd
            pn = _silu_h(jnp.dot(h, wn0h_ref[l], preferred_element_type=f32)
                         + jnp.dot(hagg, wn0g_ref[l],
                                   preferred_element_type=f32)
                         + bn0_ref[l])
            h = h + jnp.dot(pn, wn1_ref[l], preferred_element_type=f32) \
                + bn1_ref[l]

    vel = x - x0
    mean = jnp.sum(vel * nmask, axis=0, keepdims=True) / N_P
    out_ref[0] = vel - mean


def _bd(W):
    z = jnp.zeros_like(W)
    return jnp.concatenate(
        [jnp.concatenate([W, z], axis=1), jnp.concatenate([z, W], axis=1)],
        axis=0)


def _dup(b):
    return jnp.concatenate([b, b])[None, :]


def _rank1_12(wr, wa):
    z = jnp.zeros_like(wr)
    ra = jnp.concatenate([wr, z])
    rb = jnp.concatenate([z, wr])
    aa = jnp.concatenate([wa, z])
    ab = jnp.concatenate([z, wa])
    return jnp.stack([ra, ra, ra, rb, rb, rb, aa, aa, aa, ab, ab, ab])


def _wc1_12(w):
    w = w[:, 0]
    z = jnp.zeros_like(w)
    ca = jnp.concatenate([w, z])
    cb = jnp.concatenate([z, w])
    zz = jnp.zeros_like(ca)
    return jnp.stack([ca, ca, ca, cb, cb, cb, zz, zz, zz, zz, zz, zz], axis=1)


def kernel(t, xs, params, rows, cols):
    f32 = jnp.float32
    layers = params["layers"]
    half = 0.5

    embw = _dup(params["embedding"]["W"][0])
    embb = _dup(params["embedding"]["b"])
    w0h = jnp.stack([_bd(half * lp["edge_mlp0"]["W"][:H]) for lp in layers])
    w0c = jnp.stack([_bd(half * lp["edge_mlp0"]["W"][H:2 * H])
                     for lp in layers])
    wra = jnp.stack([_rank1_12(half * lp["edge_mlp0"]["W"][2 * H],
                               half * lp["edge_mlp0"]["W"][2 * H + 1])
                     for lp in layers])
    b0 = jnp.stack([_dup(half * lp["edge_mlp0"]["b"]) for lp in layers])
    w1 = jnp.stack([_bd(half * lp["edge_mlp1"]["W"]) for lp in layers])
    b1 = jnp.stack([_dup(half * lp["edge_mlp1"]["b"]) for lp in layers])
    wc0 = jnp.stack([_bd(half * lp["coord_mlp0"]["W"]) for lp in layers])
    bc0 = jnp.stack([_dup(half * lp["coord_mlp0"]["b"]) for lp in layers])
    wc1 = jnp.stack([_wc1_12(lp["coord_mlp1"]["W"]) for lp in layers])
    wn0h = jnp.stack([_bd(half * lp["node_mlp0"]["W"][:H]) for lp in layers])
    wn0g = jnp.stack([_bd(half * lp["node_mlp0"]["W"][H:]) for lp in layers])
    bn0 = jnp.stack([_dup(half * lp["node_mlp0"]["b"]) for lp in layers])
    wn1 = jnp.stack([_bd(lp["node_mlp1"]["W"]) for lp in layers])
    bn1 = jnp.stack([_dup(lp["node_mlp1"]["b"]) for lp in layers])

    a_i = jnp.arange(12)
    sel = ((a_i[:, None] < 6) & (a_i[None, :] < 6)
           & (a_i[:, None] // 3 == a_i[None, :] // 3)).astype(f32)

    r_i = jnp.arange(PAD * PAD)
    c_i = jnp.arange(2 * H)
    pij = (jnp.where(c_i[None, :] < H,
                     (r_i[:, None] // PAD) == c_i[None, :],
                     (r_i[:, None] % PAD) == (c_i[None, :] - H))
           ).astype(f32)
    jm = ((r_i[:, None] % PAD) < N_P).astype(f32) * jnp.ones((1, 12), f32)

    t3 = t.astype(f32).reshape(PAIRS, 1, 2)
    xg = xs.astype(f32).reshape(N_B, N_P, 3)
    xg = jnp.pad(xg, ((0, 0), (0, PAD - N_P), (0, 0)))
    xp = xg.reshape(PAIRS, 2, PAD, 3).transpose(0, 2, 1, 3).reshape(
        PAIRS, PAD, 6)
    xp = jnp.concatenate([xp, xp], axis=2)

    def full(a):
        return pl.BlockSpec(a.shape, lambda s: (0,) * a.ndim)

    weights = (embw, embb, w0h, w0c, wra, b0, w1, b1, wc0, bc0, wc1,
               wn0h, wn0g, bn0, wn1, bn1, sel, pij, jm)

    out = pl.pallas_call(
        _egnn_step,
        grid=(PAIRS,),
        in_specs=[
            pl.BlockSpec((1, 1, 2), lambda s: (s, 0, 0)),
            pl.BlockSpec((1, PAD, 12), lambda s: (s, 0, 0)),
        ] + [full(w) for w in weights],
        out_specs=pl.BlockSpec((1, PAD, 12), lambda s: (s, 0, 0)),
        out_shape=jax.ShapeDtypeStruct((PAIRS, PAD, 12), f32),
        compiler_params=pltpu.CompilerParams(
            dimension_semantics=("arbitrary",)),
    )(t3, xp, *weights)

    vel = out[:, :, :6].reshape(PAIRS, PAD, 2, 3).transpose(0, 2, 1, 3)
    vel = vel.reshape(N_B, PAD, 3)[:, :N_P, :]
    return vel.reshape(N_B, N_P * 3)


# speedup vs baseline: 1.1144x; 1.1144x over previous
import jax
import jax.numpy as jnp
from jax import lax
from jax.experimental import pallas as pl
from jax.experimental.pallas import tpu as pltpu

N_B = 128
N_P = 55
PAD = 64
H = 64
L = 4
PAIRS = N_B // 2


def _silu_h(u):
    t = lax.tanh(u)
    return u * t + u


def _egnn_step(t_ref, x_ref,
               embw_ref, embb_ref,
               w0h_ref, w0c_ref, wra_ref, b0_ref,
               w1_ref, b1_ref,
               wc0_ref, bc0_ref, wc1_ref,
               wn0h_ref, wn0g_ref, bn0_ref, wn1_ref, bn1_ref,
               s12_ref, pij_ref, jm_ref,
               out_ref):
    f32 = jnp.float32
    x0 = x_ref[0]
    ta = t_ref[0, 0, 0]
    tb = t_ref[0, 0, 1]

    lane = lax.broadcasted_iota(jnp.int32, (1, 2 * H), 1)
    t_row = jnp.where(lane < H, ta, tb).astype(f32)
    h_row = t_row * embw_ref[...] + embb_ref[...]
    h = jnp.broadcast_to(h_row, (PAD, 2 * H))

    nmask = (lax.broadcasted_iota(jnp.int32, (PAD, 1), 0) < N_P).astype(f32)

    S12 = s12_ref[...]
    PIJ = pij_ref[...]
    JM = jm_ref[...]

    x = x0
    for l in range(L):
        diff = (x[:, None, :] - x[None, :, :]).reshape(PAD * PAD, 12)
        dsq = diff * diff
        radial = jnp.dot(dsq, S12, preferred_element_type=f32)
        inv = lax.rsqrt(radial + 1e-8) * JM

        A = jnp.dot(h, w0h_ref[l], preferred_element_type=f32) + b0_ref[l]
        B = jnp.dot(h, w0c_ref[l], preferred_element_type=f32)
        pre0 = ((A[:, None, :] + B[None, :, :]).reshape(PAD * PAD, 2 * H)
                + jnp.dot(dsq, wra_ref[l], preferred_element_type=f32))
        e0 = _silu_h(pre0)
        ef = _silu_h(jnp.dot(e0, w1_ref[l], preferred_element_type=f32)
                     + b1_ref[l])

        c0 = _silu_h(jnp.dot(ef, wc0_ref[l], preferred_element_type=f32)
                     + bc0_ref[l])
        s12 = jnp.dot(c0, wc1_ref[l], preferred_element_type=f32)
        trans = diff * (s12 * inv)
        upd = jnp.sum(trans.reshape(PAD, PAD, 12), axis=1)
        x = x + upd

        if l < L - 1:
            hagg = jnp.sum(ef.reshape(PAD, PAD, 2 * H)[:, :N_P, :], axis=1)
            e0d = _silu_h(A + B)
            efd = _silu_h(jnp.dot(e0d, w1_ref[l], preferred_element_type=f32)
                          + b1_ref[l])
            hagg = hagg - efd
            pn = _silu_h(jnp.dot(h, wn0h_ref[l], preferred_element_type=f32)
                         + jnp.dot(hagg, wn0g_ref[l],
                                   preferred_element_type=f32)
                         + bn0_ref[l])
            h = h + jnp.dot(pn, wn1_ref[l], preferred_element_type=f32) \
                + bn1_ref[l]

    vel = x - x0
    mean = jnp.sum(vel * nmask, axis=0, keepdims=True) / N_P
    out_ref[0] = vel - mean


def _bd(W):
    z = jnp.zeros_like(W)
    return jnp.concatenate(
        [jnp.concatenate([W, z], axis=1), jnp.concatenate([z, W], axis=1)],
        axis=0)


def _dup(b):
    return jnp.concatenate([b, b])[None, :]


def _rank1_12(wr, wa):
    z = jnp.zeros_like(wr)
    ra = jnp.concatenate([wr, z])
    rb = jnp.concatenate([z, wr])
    aa = jnp.concatenate([wa, z])
    ab = jnp.concatenate([z, wa])
    return jnp.stack([ra, ra, ra, rb, rb, rb, aa, aa, aa, ab, ab, ab])


def _wc1_12(w):
    w = w[:, 0]
    z = jnp.zeros_like(w)
    ca = jnp.concatenate([w, z])
    cb = jnp.concatenate([z, w])
    zz = jnp.zeros_like(ca)
    return jnp.stack([ca, ca, ca, cb, cb, cb, zz, zz, zz, zz, zz, zz], axis=1)


def kernel(t, xs, params, rows, cols):
    f32 = jnp.float32
    layers = params["layers"]
    half = 0.5

    embw = _dup(params["embedding"]["W"][0])
    embb = _dup(params["embedding"]["b"])
    w0h = jnp.stack([_bd(half * lp["edge_mlp0"]["W"][:H]) for lp in layers])
    w0c = jnp.stack([_bd(half * lp["edge_mlp0"]["W"][H:2 * H])
                     for lp in layers])
    wra = jnp.stack([_rank1_12(half * lp["edge_mlp0"]["W"][2 * H],
                               half * lp["edge_mlp0"]["W"][2 * H + 1])
                     for lp in layers])
    b0 = jnp.stack([_dup(half * lp["edge_mlp0"]["b"]) for lp in layers])
    w1 = jnp.stack([_bd(half * lp["edge_mlp1"]["W"]) for lp in layers])
    b1 = jnp.stack([_dup(half * lp["edge_mlp1"]["b"]) for lp in layers])
    wc0 = jnp.stack([_bd(half * lp["coord_mlp0"]["W"]) for lp in layers])
    bc0 = jnp.stack([_dup(half * lp["coord_mlp0"]["b"]) for lp in layers])
    wc1 = jnp.stack([_wc1_12(lp["coord_mlp1"]["W"]) for lp in layers])
    wn0h = jnp.stack([_bd(half * lp["node_mlp0"]["W"][:H]) for lp in layers])
    wn0g = jnp.stack([_bd(half * lp["node_mlp0"]["W"][H:]) for lp in layers])
    bn0 = jnp.stack([_dup(half * lp["node_mlp0"]["b"]) for lp in layers])
    wn1 = jnp.stack([_bd(lp["node_mlp1"]["W"]) for lp in layers])
    bn1 = jnp.stack([_dup(lp["node_mlp1"]["b"]) for lp in layers])

    a_i = jnp.arange(12)
    sel = ((a_i[:, None] < 6) & (a_i[None, :] < 6)
           & (a_i[:, None] // 3 == a_i[None, :] // 3)).astype(f32)

    r_i = jnp.arange(PAD * PAD)
    c_i = jnp.arange(2 * H)
    pij = (jnp.where(c_i[None, :] < H,
                     (r_i[:, None] // PAD) == c_i[None, :],
                     (r_i[:, None] % PAD) == (c_i[None, :] - H))
           ).astype(f32)
    jm = ((r_i[:, None] % PAD) < N_P).astype(f32) * jnp.ones((1, 12), f32)

    t3 = t.astype(f32).reshape(PAIRS, 1, 2)
    xg = xs.astype(f32).reshape(N_B, N_P, 3)
    xg = jnp.pad(xg, ((0, 0), (0, PAD - N_P), (0, 0)))
    xp = xg.reshape(PAIRS, 2, PAD, 3).transpose(0, 2, 1, 3).reshape(
        PAIRS, PAD, 6)
    xp = jnp.concatenate([xp, xp], axis=2)

    def full(a):
        return pl.BlockSpec(a.shape, lambda s: (0,) * a.ndim)

    weights = (embw, embb, w0h, w0c, wra, b0, w1, b1, wc0, bc0, wc1,
               wn0h, wn0g, bn0, wn1, bn1, sel, pij, jm)

    out = pl.pallas_call(
        _egnn_step,
        grid=(PAIRS,),
        in_specs=[
            pl.BlockSpec((1, 1, 2), lambda s: (s, 0, 0)),
            pl.BlockSpec((1, PAD, 12), lambda s: (s, 0, 0)),
        ] + [full(w) for w in weights],
        out_specs=pl.BlockSpec((1, PAD, 12), lambda s: (s, 0, 0)),
        out_shape=jax.ShapeDtypeStruct((PAIRS, PAD, 12), f32),
        compiler_params=pltpu.CompilerParams(
            dimension_semantics=("arbitrary",)),
    )(t3, xp, *weights)

    vel = out[:, :, :6].reshape(PAIRS, PAD, 2, 3).transpose(0, 2, 1, 3)
    vel = vel.reshape(N_B, PAD, 3)[:, :N_P, :]
    return vel.reshape(N_B, N_P * 3)


# speedup vs baseline: 1.4331x; 1.2861x over previous
import jax
import jax.numpy as jnp
from jax import lax
from jax.experimental import pallas as pl
from jax.experimental.pallas import tpu as pltpu

N_B = 128
N_P = 55
PAD = 56
H = 64
L = 4
PAIRS = N_B // 2


def _silu_h(u):
    t = lax.tanh(u)
    return u * t + u


def _egnn_step(t_ref, x_ref,
               embw_ref, embb_ref,
               w0h_ref, w0c_ref, wra_ref, b0_ref,
               w1_ref, b1_ref,
               wc0_ref, bc0_ref, wc1_ref,
               wn0h_ref, wn0g_ref, bn0_ref, wn1_ref, bn1_ref,
               s12_ref,
               out_ref):
    f32 = jnp.float32
    x0 = x_ref[0]
    ta = t_ref[0, 0, 0]
    tb = t_ref[0, 0, 1]

    lane = lax.broadcasted_iota(jnp.int32, (1, 2 * H), 1)
    t_row = jnp.where(lane < H, ta, tb).astype(f32)
    h_row = t_row * embw_ref[...] + embb_ref[...]
    h = jnp.broadcast_to(h_row, (PAD, 2 * H))

    nmask = (lax.broadcasted_iota(jnp.int32, (PAD, 1), 0) < N_P).astype(f32)

    S12 = s12_ref[...]

    x = x0
    for l in range(L):
        diff = (x[:, None, :] - x[None, :, :]).reshape(PAD * PAD, 12)
        dsq = diff * diff
        radial = jnp.dot(dsq, S12, preferred_element_type=f32)
        inv = lax.rsqrt(radial + 1e-8)

        A = jnp.dot(h, w0h_ref[l], preferred_element_type=f32) + b0_ref[l]
        B = jnp.dot(h, w0c_ref[l], preferred_element_type=f32)
        pre0 = ((A[:, None, :] + B[None, :, :]).reshape(PAD * PAD, 2 * H)
                + jnp.dot(dsq, wra_ref[l], preferred_element_type=f32))
        e0 = _silu_h(pre0)
        ef = _silu_h(jnp.dot(e0, w1_ref[l], preferred_element_type=f32)
                     + b1_ref[l])

        c0 = _silu_h(jnp.dot(ef, wc0_ref[l], preferred_element_type=f32)
                     + bc0_ref[l])
        s12 = jnp.dot(c0, wc1_ref[l], preferred_element_type=f32)
        trans = diff * (s12 * inv)
        upd = jnp.sum(trans.reshape(PAD, PAD, 12)[:, :N_P, :], axis=1)
        x = x + upd

        if l < L - 1:
            hagg = jnp.sum(ef.reshape(PAD, PAD, 2 * H)[:, :N_P, :], axis=1)
            e0d = _silu_h(A + B)
            efd = _silu_h(jnp.dot(e0d, w1_ref[l], preferred_element_type=f32)
                          + b1_ref[l])
            hagg = hagg - efd
            pn = _silu_h(jnp.dot(h, wn0h_ref[l], preferred_element_type=f32)
                         + jnp.dot(hagg, wn0g_ref[l],
                                   preferred_element_type=f32)
                         + bn0_ref[l])
            h = h + jnp.dot(pn, wn1_ref[l], preferred_element_type=f32) \
                + bn1_ref[l]

    vel = x - x0
    mean = jnp.sum(vel * nmask, axis=0, keepdims=True) / N_P
    out_ref[0] = vel - mean


def _bd(W):
    z = jnp.zeros_like(W)
    return jnp.concatenate(
        [jnp.concatenate([W, z], axis=1), jnp.concatenate([z, W], axis=1)],
        axis=0)


def _dup(b):
    return jnp.concatenate([b, b])[None, :]


def _rank1_12(wr, wa):
    z = jnp.zeros_like(wr)
    ra = jnp.concatenate([wr, z])
    rb = jnp.concatenate([z, wr])
    aa = jnp.concatenate([wa, z])
    ab = jnp.concatenate([z, wa])
    return jnp.stack([ra, ra, ra, rb, rb, rb, aa, aa, aa, ab, ab, ab])


def _wc1_12(w):
    w = w[:, 0]
    z = jnp.zeros_like(w)
    ca = jnp.concatenate([w, z])
    cb = jnp.concatenate([z, w])
    zz = jnp.zeros_like(ca)
    return jnp.stack([ca, ca, ca, cb, cb, cb, zz, zz, zz, zz, zz, zz], axis=1)


def kernel(t, xs, params, rows, cols):
    f32 = jnp.float32
    layers = params["layers"]
    half = 0.5

    embw = _dup(params["embedding"]["W"][0])
    embb = _dup(params["embedding"]["b"])
    w0h = jnp.stack([_bd(half * lp["edge_mlp0"]["W"][:H]) for lp in layers])
    w0c = jnp.stack([_bd(half * lp["edge_mlp0"]["W"][H:2 * H])
                     for lp in layers])
    wra = jnp.stack([_rank1_12(half * lp["edge_mlp0"]["W"][2 * H],
                               half * lp["edge_mlp0"]["W"][2 * H + 1])
                     for lp in layers])
    b0 = jnp.stack([_dup(half * lp["edge_mlp0"]["b"]) for lp in layers])
    w1 = jnp.stack([_bd(half * lp["edge_mlp1"]["W"]) for lp in layers])
    b1 = jnp.stack([_dup(half * lp["edge_mlp1"]["b"]) for lp in layers])
    wc0 = jnp.stack([_bd(half * lp["coord_mlp0"]["W"]) for lp in layers])
    bc0 = jnp.stack([_dup(half * lp["coord_mlp0"]["b"]) for lp in layers])
    wc1 = jnp.stack([_wc1_12(lp["coord_mlp1"]["W"]) for lp in layers])
    wn0h = jnp.stack([_bd(half * lp["node_mlp0"]["W"][:H]) for lp in layers])
    wn0g = jnp.stack([_bd(half * lp["node_mlp0"]["W"][H:]) for lp in layers])
    bn0 = jnp.stack([_dup(half * lp["node_mlp0"]["b"]) for lp in layers])
    wn1 = jnp.stack([_bd(lp["node_mlp1"]["W"]) for lp in layers])
    bn1 = jnp.stack([_dup(lp["node_mlp1"]["b"]) for lp in layers])

    a_i = jnp.arange(12)
    sel = ((a_i[:, None] < 6) & (a_i[None, :] < 6)
           & (a_i[:, None] // 3 == a_i[None, :] // 3)).astype(f32)

    t3 = t.astype(f32).reshape(PAIRS, 1, 2)
    xg = xs.astype(f32).reshape(N_B, N_P, 3)
    xg = jnp.pad(xg, ((0, 0), (0, PAD - N_P), (0, 0)))
    xp = xg.reshape(PAIRS, 2, PAD, 3).transpose(0, 2, 1, 3).reshape(
        PAIRS, PAD, 6)
    xp = jnp.concatenate([xp, xp], axis=2)

    def full(a):
        return pl.BlockSpec(a.shape, lambda s: (0,) * a.ndim)

    weights = (embw, embb, w0h, w0c, wra, b0, w1, b1, wc0, bc0, wc1,
               wn0h, wn0g, bn0, wn1, bn1, sel)

    out = pl.pallas_call(
        _egnn_step,
        grid=(PAIRS,),
        in_specs=[
            pl.BlockSpec((1, 1, 2), lambda s: (s, 0, 0)),
            pl.BlockSpec((1, PAD, 12), lambda s: (s, 0, 0)),
        ] + [full(w) for w in weights],
        out_specs=pl.BlockSpec((1, PAD, 12), lambda s: (s, 0, 0)),
        out_shape=jax.ShapeDtypeStruct((PAIRS, PAD, 12), f32),
        compiler_params=pltpu.CompilerParams(
            dimension_semantics=("arbitrary",)),
    )(t3, xp, *weights)

    vel = out[:, :, :6].reshape(PAIRS, PAD, 2, 3).transpose(0, 2, 1, 3)
    vel = vel.reshape(N_B, PAD, 3)[:, :N_P, :]
    return vel.reshape(N_B, N_P * 3)


# speedup vs baseline: 1.5083x; 1.0524x over previous
import jax
import jax.numpy as jnp
from jax import lax
from jax.experimental import pallas as pl
from jax.experimental.pallas import tpu as pltpu

N_B = 128
N_P = 55
PAD = 56
H = 64
L = 4
PAIRS = N_B // 2


def _silu_h(u):
    t = lax.tanh(u)
    return u * t + u


def _egnn_step(t_ref, x_ref,
               embw_ref, embb_ref,
               w0h_ref, w0c_ref, wra_ref, b0_ref,
               w1_ref, b1_ref,
               wc0_ref, bc0_ref, wc1_ref,
               wn0h_ref, wn0g_ref, bn0_ref, wn1_ref, bn1_ref,
               s12_ref,
               out_ref):
    f32 = jnp.float32
    x0 = x_ref[0]
    ta = t_ref[0, 0, 0]
    tb = t_ref[0, 0, 1]

    lane = lax.broadcasted_iota(jnp.int32, (1, 2 * H), 1)
    t_row = jnp.where(lane < H, ta, tb).astype(f32)
    h_row = t_row * embw_ref[...] + embb_ref[...]
    h = jnp.broadcast_to(h_row, (PAD, 2 * H))

    nmask = (lax.broadcasted_iota(jnp.int32, (PAD, 1), 0) < N_P).astype(f32)

    S12 = s12_ref[...]

    x = x0
    for l in range(L):
        diff = (x[None, :, :] - x[:, None, :]).reshape(PAD * PAD, 12)
        dsq = diff * diff
        radial = jnp.dot(dsq, S12, preferred_element_type=f32)
        inv = lax.rsqrt(radial + 1e-8)

        A = jnp.dot(h, w0h_ref[l], preferred_element_type=f32) + b0_ref[l]
        B = jnp.dot(h, w0c_ref[l], preferred_element_type=f32)
        pre0 = ((A[None, :, :] + B[:, None, :]).reshape(PAD * PAD, 2 * H)
                + jnp.dot(dsq, wra_ref[l], preferred_element_type=f32))
        e0 = _silu_h(pre0)
        ef = _silu_h(jnp.dot(e0, w1_ref[l], preferred_element_type=f32)
                     + b1_ref[l])

        c0 = _silu_h(jnp.dot(ef, wc0_ref[l], preferred_element_type=f32)
                     + bc0_ref[l])
        s12 = jnp.dot(c0, wc1_ref[l], preferred_element_type=f32)
        trans = diff * (s12 * inv)
        upd = jnp.sum(trans.reshape(PAD, PAD, 12)[:N_P, :, :], axis=0)
        x = x + upd

        if l < L - 1:
            hagg = jnp.sum(ef.reshape(PAD, PAD, 2 * H)[:N_P, :, :], axis=0)
            e0d = _silu_h(A + B)
            efd = _silu_h(jnp.dot(e0d, w1_ref[l], preferred_element_type=f32)
                          + b1_ref[l])
            hagg = hagg - efd
            pn = _silu_h(jnp.dot(h, wn0h_ref[l], preferred_element_type=f32)
                         + jnp.dot(hagg, wn0g_ref[l],
                                   preferred_element_type=f32)
                         + bn0_ref[l])
            h = h + jnp.dot(pn, wn1_ref[l], preferred_element_type=f32) \
                + bn1_ref[l]

    vel = x - x0
    mean = jnp.sum(vel * nmask, axis=0, keepdims=True) / N_P
    out_ref[0] = vel - mean


def _bd(W):
    z = jnp.zeros_like(W)
    return jnp.concatenate(
        [jnp.concatenate([W, z], axis=1), jnp.concatenate([z, W], axis=1)],
        axis=0)


def _dup(b):
    return jnp.concatenate([b, b])[None, :]


def _rank1_12(wr, wa):
    z = jnp.zeros_like(wr)
    ra = jnp.concatenate([wr, z])
    rb = jnp.concatenate([z, wr])
    aa = jnp.concatenate([wa, z])
    ab = jnp.concatenate([z, wa])
    return jnp.stack([ra, ra, ra, rb, rb, rb, aa, aa, aa, ab, ab, ab])


def _wc1_12(w):
    w = w[:, 0]
    z = jnp.zeros_like(w)
    ca = jnp.concatenate([w, z])
    cb = jnp.concatenate([z, w])
    zz = jnp.zeros_like(ca)
    return jnp.stack([ca, ca, ca, cb, cb, cb, zz, zz, zz, zz, zz, zz], axis=1)


def kernel(t, xs, params, rows, cols):
    f32 = jnp.float32
    layers = params["layers"]
    half = 0.5

    embw = _dup(params["embedding"]["W"][0])
    embb = _dup(params["embedding"]["b"])
    w0h = jnp.stack([_bd(half * lp["edge_mlp0"]["W"][:H]) for lp in layers])
    w0c = jnp.stack([_bd(half * lp["edge_mlp0"]["W"][H:2 * H])
                     for lp in layers])
    wra = jnp.stack([_rank1_12(half * lp["edge_mlp0"]["W"][2 * H],
                               half * lp["edge_mlp0"]["W"][2 * H + 1])
                     for lp in layers])
    b0 = jnp.stack([_dup(half * lp["edge_mlp0"]["b"]) for lp in layers])
    w1 = jnp.stack([_bd(half * lp["edge_mlp1"]["W"]) for lp in layers])
    b1 = jnp.stack([_dup(half * lp["edge_mlp1"]["b"]) for lp in layers])
    wc0 = jnp.stack([_bd(half * lp["coord_mlp0"]["W"]) for lp in layers])
    bc0 = jnp.stack([_dup(half * lp["coord_mlp0"]["b"]) for lp in layers])
    wc1 = jnp.stack([_wc1_12(lp["coord_mlp1"]["W"]) for lp in layers])
    wn0h = jnp.stack([_bd(half * lp["node_mlp0"]["W"][:H]) for lp in layers])
    wn0g = jnp.stack([_bd(half * lp["node_mlp0"]["W"][H:]) for lp in layers])
    bn0 = jnp.stack([_dup(half * lp["node_mlp0"]["b"]) for lp in layers])
    wn1 = jnp.stack([_bd(lp["node_mlp1"]["W"]) for lp in layers])
    bn1 = jnp.stack([_dup(lp["node_mlp1"]["b"]) for lp in layers])

    a_i = jnp.arange(12)
    sel = ((a_i[:, None] < 6) & (a_i[None, :] < 6)
           & (a_i[:, None] // 3 == a_i[None, :] // 3)).astype(f32)

    t3 = t.astype(f32).reshape(PAIRS, 1, 2)
    xg = xs.astype(f32).reshape(N_B, N_P, 3)
    xg = jnp.pad(xg, ((0, 0), (0, PAD - N_P), (0, 0)))
    xp = xg.reshape(PAIRS, 2, PAD, 3).transpose(0, 2, 1, 3).reshape(
        PAIRS, PAD, 6)
    xp = jnp.concatenate([xp, xp], axis=2)

    def full(a):
        return pl.BlockSpec(a.shape, lambda s: (0,) * a.ndim)

    weights = (embw, embb, w0h, w0c, wra, b0, w1, b1, wc0, bc0, wc1,
               wn0h, wn0g, bn0, wn1, bn1, sel)

    out = pl.pallas_call(
        _egnn_step,
        grid=(PAIRS,),
        in_specs=[
            pl.BlockSpec((1, 1, 2), lambda s: (s, 0, 0)),
            pl.BlockSpec((1, PAD, 12), lambda s: (s, 0, 0)),
        ] + [full(w) for w in weights],
        out_specs=pl.BlockSpec((1, PAD, 12), lambda s: (s, 0, 0)),
        out_shape=jax.ShapeDtypeStruct((PAIRS, PAD, 12), f32),
        compiler_params=pltpu.CompilerParams(
            dimension_semantics=("arbitrary",)),
    )(t3, xp, *weights)

    vel = out[:, :, :6].reshape(PAIRS, PAD, 2, 3).transpose(0, 2, 1, 3)
    vel = vel.reshape(N_B, PAD, 3)[:, :N_P, :]
    return vel.reshape(N_B, N_P * 3)


# speedup vs baseline: 1.5126x; 1.0028x over previous
import jax
import jax.numpy as jnp
from jax import lax
from jax.experimental import pallas as pl
from jax.experimental.pallas import tpu as pltpu

N_B = 128
N_P = 55
PAD = 56
H = 64
L = 4
PAIRS = N_B // 2


def _silu_h(u):
    t = lax.tanh(u)
    return u * t + u


def _egnn_step(t_ref, x_ref,
               embw_ref, embb_ref,
               w0h_ref, w0c_ref, wra_ref, b0_ref,
               w1_ref, b1_ref,
               wc0_ref, bc0_ref, wc1_ref,
               wn0h_ref, wn0g_ref, bn0_ref, wn1_ref, bn1_ref,
               s12_ref,
               out_ref):
    f32 = jnp.float32
    x0 = x_ref[0]
    ta = t_ref[0, 0, 0]
    tb = t_ref[0, 0, 1]

    lane = lax.broadcasted_iota(jnp.int32, (1, 2 * H), 1)
    t_row = jnp.where(lane < H, ta, tb).astype(f32)
    h_row = t_row * embw_ref[...] + embb_ref[...]
    h = jnp.broadcast_to(h_row, (PAD, 2 * H))

    nmask = (lax.broadcasted_iota(jnp.int32, (PAD, 1), 0) < N_P).astype(f32)

    S12 = s12_ref[...]

    x = x0
    for l in range(L):
        diff = (x[None, :, :] - x[:, None, :]).reshape(PAD * PAD, 12)
        dsq = diff * diff
        radial = jnp.dot(dsq, S12, preferred_element_type=f32)
        inv = lax.rsqrt(radial + 1e-8)

        if l == 0:
            ab = (jnp.dot(h_row, w0h_ref[l], preferred_element_type=f32)
                  + jnp.dot(h_row, w0c_ref[l], preferred_element_type=f32)
                  + b0_ref[l])
            pre0 = jnp.dot(dsq, wra_ref[l], preferred_element_type=f32) + ab
        else:
            A = jnp.dot(h, w0h_ref[l], preferred_element_type=f32) \
                + b0_ref[l]
            B = jnp.dot(h, w0c_ref[l], preferred_element_type=f32)
            ab = A + B
            pre0 = ((A[None, :, :] + B[:, None, :]).reshape(PAD * PAD, 2 * H)
                    + jnp.dot(dsq, wra_ref[l], preferred_element_type=f32))
        e0 = _silu_h(pre0)
        ef = _silu_h(jnp.dot(e0, w1_ref[l], preferred_element_type=f32)
                     + b1_ref[l])

        c0 = _silu_h(jnp.dot(ef, wc0_ref[l], preferred_element_type=f32)
                     + bc0_ref[l])
        s12 = jnp.dot(c0, wc1_ref[l], preferred_element_type=f32)
        trans = diff * (s12 * inv)
        upd = jnp.sum(trans.reshape(PAD, PAD, 12)[:N_P, :, :], axis=0)
        x = x + upd

        if l < L - 1:
            hagg = jnp.sum(ef.reshape(PAD, PAD, 2 * H)[:N_P, :, :], axis=0)
            e0d = _silu_h(ab)
            efd = _silu_h(jnp.dot(e0d, w1_ref[l], preferred_element_type=f32)
                          + b1_ref[l])
            hagg = hagg - efd
            pn = _silu_h(jnp.dot(h, wn0h_ref[l], preferred_element_type=f32)
                         + jnp.dot(hagg, wn0g_ref[l],
                                   preferred_element_type=f32)
                         + bn0_ref[l])
            h = h + jnp.dot(pn, wn1_ref[l], preferred_element_type=f32) \
                + bn1_ref[l]

    vel = x - x0
    mean = jnp.sum(vel * nmask, axis=0, keepdims=True) / N_P
    out_ref[0] = vel - mean


def _bd(W):
    z = jnp.zeros_like(W)
    return jnp.concatenate(
        [jnp.concatenate([W, z], axis=1), jnp.concatenate([z, W], axis=1)],
        axis=0)


def _dup(b):
    return jnp.concatenate([b, b])[None, :]


def _rank1_12(wr, wa):
    z = jnp.zeros_like(wr)
    ra = jnp.concatenate([wr, z])
    rb = jnp.concatenate([z, wr])
    aa = jnp.concatenate([wa, z])
    ab = jnp.concatenate([z, wa])
    return jnp.stack([ra, ra, ra, rb, rb, rb, aa, aa, aa, ab, ab, ab])


def _wc1_12(w):
    w = w[:, 0]
    z = jnp.zeros_like(w)
    ca = jnp.concatenate([w, z])
    cb = jnp.concatenate([z, w])
    zz = jnp.zeros_like(ca)
    return jnp.stack([ca, ca, ca, cb, cb, cb, zz, zz, zz, zz, zz, zz], axis=1)


def kernel(t, xs, params, rows, cols):
    f32 = jnp.float32
    layers = params["layers"]
    half = 0.5

    embw = _dup(params["embedding"]["W"][0])
    embb = _dup(params["embedding"]["b"])
    w0h = jnp.stack([_bd(half * lp["edge_mlp0"]["W"][:H]) for lp in layers])
    w0c = jnp.stack([_bd(half * lp["edge_mlp0"]["W"][H:2 * H])
                     for lp in layers])
    wra = jnp.stack([_rank1_12(half * lp["edge_mlp0"]["W"][2 * H],
                               half * lp["edge_mlp0"]["W"][2 * H + 1])
                     for lp in layers])
    b0 = jnp.stack([_dup(half * lp["edge_mlp0"]["b"]) for lp in layers])
    w1 = jnp.stack([_bd(half * lp["edge_mlp1"]["W"]) for lp in layers])
    b1 = jnp.stack([_dup(half * lp["edge_mlp1"]["b"]) for lp in layers])
    wc0 = jnp.stack([_bd(half * lp["coord_mlp0"]["W"]) for lp in layers])
    bc0 = jnp.stack([_dup(half * lp["coord_mlp0"]["b"]) for lp in layers])
    wc1 = jnp.stack([_wc1_12(lp["coord_mlp1"]["W"]) for lp in layers])
    wn0h = jnp.stack([_bd(half * lp["node_mlp0"]["W"][:H]) for lp in layers])
    wn0g = jnp.stack([_bd(half * lp["node_mlp0"]["W"][H:]) for lp in layers])
    bn0 = jnp.stack([_dup(half * lp["node_mlp0"]["b"]) for lp in layers])
    wn1 = jnp.stack([_bd(lp["node_mlp1"]["W"]) for lp in layers])
    bn1 = jnp.stack([_dup(lp["node_mlp1"]["b"]) for lp in layers])

    a_i = jnp.arange(12)
    sel = ((a_i[:, None] < 6) & (a_i[None, :] < 6)
           & (a_i[:, None] // 3 == a_i[None, :] // 3)).astype(f32)

    t3 = t.astype(f32).reshape(PAIRS, 1, 2)
    xg = xs.astype(f32).reshape(N_B, N_P, 3)
    xg = jnp.pad(xg, ((0, 0), (0, PAD - N_P), (0, 0)))
    xp = xg.reshape(PAIRS, 2, PAD, 3).transpose(0, 2, 1, 3).reshape(
        PAIRS, PAD, 6)
    xp = jnp.concatenate([xp, xp], axis=2)

    def full(a):
        return pl.BlockSpec(a.shape, lambda s: (0,) * a.ndim)

    weights = (embw, embb, w0h, w0c, wra, b0, w1, b1, wc0, bc0, wc1,
               wn0h, wn0g, bn0, wn1, bn1, sel)

    out = pl.pallas_call(
        _egnn_step,
        grid=(PAIRS,),
        in_specs=[
            pl.BlockSpec((1, 1, 2), lambda s: (s, 0, 0)),
            pl.BlockSpec((1, PAD, 12), lambda s: (s, 0, 0)),
        ] + [full(w) for w in weights],
        out_specs=pl.BlockSpec((1, PAD, 12), lambda s: (s, 0, 0)),
        out_shape=jax.ShapeDtypeStruct((PAIRS, PAD, 12), f32),
        compiler_params=pltpu.CompilerParams(
            dimension_semantics=("arbitrary",)),
    )(t3, xp, *weights)

    vel = out[:, :, :6].reshape(PAIRS, PAD, 2, 3).transpose(0, 2, 1, 3)
    vel = vel.reshape(N_B, PAD, 3)[:, :N_P, :]
    return vel.reshape(N_B, N_P * 3)


# speedup vs baseline: 1.5557x; 1.0285x over previous
import jax
import jax.numpy as jnp
from jax import lax
from jax.experimental import pallas as pl
from jax.experimental.pallas import tpu as pltpu

N_B = 128
N_P = 55
PAD = 56
H = 64
L = 4
PAIRS = N_B // 2
GP = 2


def _silu_h(u):
    t = lax.tanh(u)
    return u * t + u


def _egnn_step(t_ref, x_ref,
               embw_ref, embb_ref,
               w0h_ref, w0c_ref, wra_ref, b0_ref,
               w1_ref, b1_ref,
               wc0_ref, bc0_ref, wc1_ref,
               wn0h_ref, wn0g_ref, bn0_ref, wn1_ref, bn1_ref,
               s12_ref,
               out_ref):
    for p in range(GP):
        _egnn_pair(p, t_ref, x_ref,
                   embw_ref, embb_ref,
                   w0h_ref, w0c_ref, wra_ref, b0_ref,
                   w1_ref, b1_ref,
                   wc0_ref, bc0_ref, wc1_ref,
                   wn0h_ref, wn0g_ref, bn0_ref, wn1_ref, bn1_ref,
                   s12_ref, out_ref)


def _egnn_pair(p, t_ref, x_ref,
               embw_ref, embb_ref,
               w0h_ref, w0c_ref, wra_ref, b0_ref,
               w1_ref, b1_ref,
               wc0_ref, bc0_ref, wc1_ref,
               wn0h_ref, wn0g_ref, bn0_ref, wn1_ref, bn1_ref,
               s12_ref, out_ref):
    f32 = jnp.float32
    x0 = x_ref[p]
    ta = t_ref[p, 0, 0]
    tb = t_ref[p, 0, 1]

    lane = lax.broadcasted_iota(jnp.int32, (1, 2 * H), 1)
    t_row = jnp.where(lane < H, ta, tb).astype(f32)
    h_row = t_row * embw_ref[...] + embb_ref[...]
    h = jnp.broadcast_to(h_row, (PAD, 2 * H))

    nmask = (lax.broadcasted_iota(jnp.int32, (PAD, 1), 0) < N_P).astype(f32)

    S12 = s12_ref[...]

    x = x0
    for l in range(L):
        diff = (x[None, :, :] - x[:, None, :]).reshape(PAD * PAD, 12)
        dsq = diff * diff
        radial = jnp.dot(dsq, S12, preferred_element_type=f32)
        inv = lax.rsqrt(radial + 1e-8)

        if l == 0:
            ab = (jnp.dot(h_row, w0h_ref[l], preferred_element_type=f32)
                  + jnp.dot(h_row, w0c_ref[l], preferred_element_type=f32)
                  + b0_ref[l])
            pre0 = jnp.dot(dsq, wra_ref[l], preferred_element_type=f32) + ab
        else:
            A = jnp.dot(h, w0h_ref[l], preferred_element_type=f32) \
                + b0_ref[l]
            B = jnp.dot(h, w0c_ref[l], preferred_element_type=f32)
            ab = A + B
            pre0 = ((A[None, :, :] + B[:, None, :]).reshape(PAD * PAD, 2 * H)
                    + jnp.dot(dsq, wra_ref[l], preferred_element_type=f32))
        e0 = _silu_h(pre0)
        ef = _silu_h(jnp.dot(e0, w1_ref[l], preferred_element_type=f32)
                     + b1_ref[l])

        c0 = _silu_h(jnp.dot(ef, wc0_ref[l], preferred_element_type=f32)
                     + bc0_ref[l])
        s12 = jnp.dot(c0, wc1_ref[l], preferred_element_type=f32)
        trans = diff * (s12 * inv)
        upd = jnp.sum(trans.reshape(PAD, PAD, 12)[:N_P, :, :], axis=0)
        x = x + upd

        if l < L - 1:
            hagg = jnp.sum(ef.reshape(PAD, PAD, 2 * H)[:N_P, :, :], axis=0)
            e0d = _silu_h(ab)
            efd = _silu_h(jnp.dot(e0d, w1_ref[l], preferred_element_type=f32)
                          + b1_ref[l])
            hagg = hagg - efd
            pn = _silu_h(jnp.dot(h, wn0h_ref[l], preferred_element_type=f32)
                         + jnp.dot(hagg, wn0g_ref[l],
                                   preferred_element_type=f32)
                         + bn0_ref[l])
            h = h + jnp.dot(pn, wn1_ref[l], preferred_element_type=f32) \
                + bn1_ref[l]

    vel = x - x0
    mean = jnp.sum(vel * nmask, axis=0, keepdims=True) / N_P
    out_ref[p] = vel - mean


def _bd(W):
    z = jnp.zeros_like(W)
    return jnp.concatenate(
        [jnp.concatenate([W, z], axis=1), jnp.concatenate([z, W], axis=1)],
        axis=0)


def _dup(b):
    return jnp.concatenate([b, b])[None, :]


def _rank1_12(wr, wa):
    z = jnp.zeros_like(wr)
    ra = jnp.concatenate([wr, z])
    rb = jnp.concatenate([z, wr])
    aa = jnp.concatenate([wa, z])
    ab = jnp.concatenate([z, wa])
    return jnp.stack([ra, ra, ra, rb, rb, rb, aa, aa, aa, ab, ab, ab])


def _wc1_12(w):
    w = w[:, 0]
    z = jnp.zeros_like(w)
    ca = jnp.concatenate([w, z])
    cb = jnp.concatenate([z, w])
    zz = jnp.zeros_like(ca)
    return jnp.stack([ca, ca, ca, cb, cb, cb, zz, zz, zz, zz, zz, zz], axis=1)


def kernel(t, xs, params, rows, cols):
    f32 = jnp.float32
    layers = params["layers"]
    half = 0.5

    embw = _dup(params["embedding"]["W"][0])
    embb = _dup(params["embedding"]["b"])
    w0h = jnp.stack([_bd(half * lp["edge_mlp0"]["W"][:H]) for lp in layers])
    w0c = jnp.stack([_bd(half * lp["edge_mlp0"]["W"][H:2 * H])
                     for lp in layers])
    wra = jnp.stack([_rank1_12(half * lp["edge_mlp0"]["W"][2 * H],
                               half * lp["edge_mlp0"]["W"][2 * H + 1])
                     for lp in layers])
    b0 = jnp.stack([_dup(half * lp["edge_mlp0"]["b"]) for lp in layers])
    w1 = jnp.stack([_bd(half * lp["edge_mlp1"]["W"]) for lp in layers])
    b1 = jnp.stack([_dup(half * lp["edge_mlp1"]["b"]) for lp in layers])
    wc0 = jnp.stack([_bd(half * lp["coord_mlp0"]["W"]) for lp in layers])
    bc0 = jnp.stack([_dup(half * lp["coord_mlp0"]["b"]) for lp in layers])
    wc1 = jnp.stack([_wc1_12(lp["coord_mlp1"]["W"]) for lp in layers])
    wn0h = jnp.stack([_bd(half * lp["node_mlp0"]["W"][:H]) for lp in layers])
    wn0g = jnp.stack([_bd(half * lp["node_mlp0"]["W"][H:]) for lp in layers])
    bn0 = jnp.stack([_dup(half * lp["node_mlp0"]["b"]) for lp in layers])
    wn1 = jnp.stack([_bd(lp["node_mlp1"]["W"]) for lp in layers])
    bn1 = jnp.stack([_dup(lp["node_mlp1"]["b"]) for lp in layers])

    a_i = jnp.arange(12)
    sel = ((a_i[:, None] < 6) & (a_i[None, :] < 6)
           & (a_i[:, None] // 3 == a_i[None, :] // 3)).astype(f32)

    t3 = t.astype(f32).reshape(PAIRS, 1, 2)
    xg = xs.astype(f32).reshape(N_B, N_P, 3)
    xg = jnp.pad(xg, ((0, 0), (0, PAD - N_P), (0, 0)))
    xp = xg.reshape(PAIRS, 2, PAD, 3).transpose(0, 2, 1, 3).reshape(
        PAIRS, PAD, 6)
    xp = jnp.concatenate([xp, xp], axis=2)

    def full(a):
        return pl.BlockSpec(a.shape, lambda s: (0,) * a.ndim)

    weights = (embw, embb, w0h, w0c, wra, b0, w1, b1, wc0, bc0, wc1,
               wn0h, wn0g, bn0, wn1, bn1, sel)

    out = pl.pallas_call(
        _egnn_step,
        grid=(PAIRS // GP,),
        in_specs=[
            pl.BlockSpec((GP, 1, 2), lambda s: (s, 0, 0)),
            pl.BlockSpec((GP, PAD, 12), lambda s: (s, 0, 0)),
        ] + [full(w) for w in weights],
        out_specs=pl.BlockSpec((GP, PAD, 12), lambda s: (s, 0, 0)),
        out_shape=jax.ShapeDtypeStruct((PAIRS, PAD, 12), f32),
        compiler_params=pltpu.CompilerParams(
            dimension_semantics=("arbitrary",)),
    )(t3, xp, *weights)

    vel = out[:, :, :6].reshape(PAIRS, PAD, 2, 3).transpose(0, 2, 1, 3)
    vel = vel.reshape(N_B, PAD, 3)[:, :N_P, :]
    return vel.reshape(N_B, N_P * 3)


# speedup vs baseline: 1.5771x; 1.0138x over previous
import jax
import jax.numpy as jnp
from jax import lax
from jax.experimental import pallas as pl
from jax.experimental.pallas import tpu as pltpu

N_B = 128
N_P = 55
PAD = 56
H = 64
L = 4
PAIRS = N_B // 2
GP = 4


def _silu_h(u):
    t = lax.tanh(u)
    return u * t + u


def _egnn_step(t_ref, x_ref,
               embw_ref, embb_ref,
               w0h_ref, w0c_ref, wra_ref, b0_ref,
               w1_ref, b1_ref,
               wc0_ref, bc0_ref, wc1_ref,
               wn0h_ref, wn0g_ref, bn0_ref, wn1_ref, bn1_ref,
               s12_ref,
               out_ref):
    for p in range(GP):
        _egnn_pair(p, t_ref, x_ref,
                   embw_ref, embb_ref,
                   w0h_ref, w0c_ref, wra_ref, b0_ref,
                   w1_ref, b1_ref,
                   wc0_ref, bc0_ref, wc1_ref,
                   wn0h_ref, wn0g_ref, bn0_ref, wn1_ref, bn1_ref,
                   s12_ref, out_ref)


def _egnn_pair(p, t_ref, x_ref,
               embw_ref, embb_ref,
               w0h_ref, w0c_ref, wra_ref, b0_ref,
               w1_ref, b1_ref,
               wc0_ref, bc0_ref, wc1_ref,
               wn0h_ref, wn0g_ref, bn0_ref, wn1_ref, bn1_ref,
               s12_ref, out_ref):
    f32 = jnp.float32
    x0 = x_ref[p]
    ta = t_ref[p, 0, 0]
    tb = t_ref[p, 0, 1]

    lane = lax.broadcasted_iota(jnp.int32, (1, 2 * H), 1)
    t_row = jnp.where(lane < H, ta, tb).astype(f32)
    h_row = t_row * embw_ref[...] + embb_ref[...]
    h = jnp.broadcast_to(h_row, (PAD, 2 * H))

    nmask = (lax.broadcasted_iota(jnp.int32, (PAD, 1), 0) < N_P).astype(f32)

    S12 = s12_ref[...]

    x = x0
    for l in range(L):
        diff = (x[None, :, :] - x[:, None, :]).reshape(PAD * PAD, 12)
        dsq = diff * diff
        radial = jnp.dot(dsq, S12, preferred_element_type=f32)
        inv = lax.rsqrt(radial + 1e-8)

        if l == 0:
            ab = (jnp.dot(h_row, w0h_ref[l], preferred_element_type=f32)
                  + jnp.dot(h_row, w0c_ref[l], preferred_element_type=f32)
                  + b0_ref[l])
            pre0 = jnp.dot(dsq, wra_ref[l], preferred_element_type=f32) + ab
        else:
            A = jnp.dot(h, w0h_ref[l], preferred_element_type=f32) \
                + b0_ref[l]
            B = jnp.dot(h, w0c_ref[l], preferred_element_type=f32)
            ab = A + B
            pre0 = ((A[None, :, :] + B[:, None, :]).reshape(PAD * PAD, 2 * H)
                    + jnp.dot(dsq, wra_ref[l], preferred_element_type=f32))
        e0 = _silu_h(pre0)
        ef = _silu_h(jnp.dot(e0, w1_ref[l], preferred_element_type=f32)
                     + b1_ref[l])

        c0 = _silu_h(jnp.dot(ef, wc0_ref[l], preferred_element_type=f32)
                     + bc0_ref[l])
        s12 = jnp.dot(c0, wc1_ref[l], preferred_element_type=f32)
        trans = diff * (s12 * inv)
        upd = jnp.sum(trans.reshape(PAD, PAD, 12)[:N_P, :, :], axis=0)
        x = x + upd

        if l < L - 1:
            hagg = jnp.sum(ef.reshape(PAD, PAD, 2 * H)[:N_P, :, :], axis=0)
            e0d = _silu_h(ab)
            efd = _silu_h(jnp.dot(e0d, w1_ref[l], preferred_element_type=f32)
                          + b1_ref[l])
            hagg = hagg - efd
            pn = _silu_h(jnp.dot(h, wn0h_ref[l], preferred_element_type=f32)
                         + jnp.dot(hagg, wn0g_ref[l],
                                   preferred_element_type=f32)
                         + bn0_ref[l])
            h = h + jnp.dot(pn, wn1_ref[l], preferred_element_type=f32) \
                + bn1_ref[l]

    vel = x - x0
    mean = jnp.sum(vel * nmask, axis=0, keepdims=True) / N_P
    out_ref[p] = vel - mean


def _bd(W):
    z = jnp.zeros_like(W)
    return jnp.concatenate(
        [jnp.concatenate([W, z], axis=1), jnp.concatenate([z, W], axis=1)],
        axis=0)


def _dup(b):
    return jnp.concatenate([b, b])[None, :]


def _rank1_12(wr, wa):
    z = jnp.zeros_like(wr)
    ra = jnp.concatenate([wr, z])
    rb = jnp.concatenate([z, wr])
    aa = jnp.concatenate([wa, z])
    ab = jnp.concatenate([z, wa])
    return jnp.stack([ra, ra, ra, rb, rb, rb, aa, aa, aa, ab, ab, ab])


def _wc1_12(w):
    w = w[:, 0]
    z = jnp.zeros_like(w)
    ca = jnp.concatenate([w, z])
    cb = jnp.concatenate([z, w])
    zz = jnp.zeros_like(ca)
    return jnp.stack([ca, ca, ca, cb, cb, cb, zz, zz, zz, zz, zz, zz], axis=1)


def kernel(t, xs, params, rows, cols):
    f32 = jnp.float32
    layers = params["layers"]
    half = 0.5

    embw = _dup(params["embedding"]["W"][0])
    embb = _dup(params["embedding"]["b"])
    w0h = jnp.stack([_bd(half * lp["edge_mlp0"]["W"][:H]) for lp in layers])
    w0c = jnp.stack([_bd(half * lp["edge_mlp0"]["W"][H:2 * H])
                     for lp in layers])
    wra = jnp.stack([_rank1_12(half * lp["edge_mlp0"]["W"][2 * H],
                               half * lp["edge_mlp0"]["W"][2 * H + 1])
                     for lp in layers])
    b0 = jnp.stack([_dup(half * lp["edge_mlp0"]["b"]) for lp in layers])
    w1 = jnp.stack([_bd(half * lp["edge_mlp1"]["W"]) for lp in layers])
    b1 = jnp.stack([_dup(half * lp["edge_mlp1"]["b"]) for lp in layers])
    wc0 = jnp.stack([_bd(half * lp["coord_mlp0"]["W"]) for lp in layers])
    bc0 = jnp.stack([_dup(half * lp["coord_mlp0"]["b"]) for lp in layers])
    wc1 = jnp.stack([_wc1_12(lp["coord_mlp1"]["W"]) for lp in layers])
    wn0h = jnp.stack([_bd(half * lp["node_mlp0"]["W"][:H]) for lp in layers])
    wn0g = jnp.stack([_bd(half * lp["node_mlp0"]["W"][H:]) for lp in layers])
    bn0 = jnp.stack([_dup(half * lp["node_mlp0"]["b"]) for lp in layers])
    wn1 = jnp.stack([_bd(lp["node_mlp1"]["W"]) for lp in layers])
    bn1 = jnp.stack([_dup(lp["node_mlp1"]["b"]) for lp in layers])

    a_i = jnp.arange(12)
    sel = ((a_i[:, None] < 6) & (a_i[None, :] < 6)
           & (a_i[:, None] // 3 == a_i[None, :] // 3)).astype(f32)

    t3 = t.astype(f32).reshape(PAIRS, 1, 2)
    xg = xs.astype(f32).reshape(N_B, N_P, 3)
    xg = jnp.pad(xg, ((0, 0), (0, PAD - N_P), (0, 0)))
    xp = xg.reshape(PAIRS, 2, PAD, 3).transpose(0, 2, 1, 3).reshape(
        PAIRS, PAD, 6)
    xp = jnp.concatenate([xp, xp], axis=2)

    def full(a):
        return pl.BlockSpec(a.shape, lambda s: (0,) * a.ndim)

    weights = (embw, embb, w0h, w0c, wra, b0, w1, b1, wc0, bc0, wc1,
               wn0h, wn0g, bn0, wn1, bn1, sel)

    out = pl.pallas_call(
        _egnn_step,
        grid=(PAIRS // GP,),
        in_specs=[
            pl.BlockSpec((GP, 1, 2), lambda s: (s, 0, 0)),
            pl.BlockSpec((GP, PAD, 12), lambda s: (s, 0, 0)),
        ] + [full(w) for w in weights],
        out_specs=pl.BlockSpec((GP, PAD, 12), lambda s: (s, 0, 0)),
        out_shape=jax.ShapeDtypeStruct((PAIRS, PAD, 12), f32),
        compiler_params=pltpu.CompilerParams(
            dimension_semantics=("arbitrary",)),
    )(t3, xp, *weights)

    vel = out[:, :, :6].reshape(PAIRS, PAD, 2, 3).transpose(0, 2, 1, 3)
    vel = vel.reshape(N_B, PAD, 3)[:, :N_P, :]
    return vel.reshape(N_B, N_P * 3)
